# baseline (speedup 1.0000x reference)
import jax
import jax.numpy as jnp
from jax.experimental import pallas as pl

_BM = 256


def _mm_kernel(xt_ref, w_ref, b_ref, o_ref):
    acc = jax.lax.dot_general(
        w_ref[...], xt_ref[...],
        dimension_numbers=(((1,), (0,)), ((), ())),
        preferred_element_type=jnp.float32,
    )
    o_ref[...] = acc + b_ref[...]


def kernel(x, W, bias):
    n, k = W.shape
    b = x.shape[0]
    xt = x.T
    b2 = bias.reshape(n, 1)
    grid = (n // _BM,)
    yt = pl.pallas_call(
        _mm_kernel,
        grid=grid,
        in_specs=[
            pl.BlockSpec((k, b), lambda i: (0, 0)),
            pl.BlockSpec((_BM, k), lambda i: (i, 0)),
            pl.BlockSpec((_BM, 1), lambda i: (i, 0)),
        ],
        out_specs=pl.BlockSpec((_BM, b), lambda i: (i, 0)),
        out_shape=jax.ShapeDtypeStruct((n, b), jnp.float32),
    )(xt, W, b2)
    return yt.T


# speedup vs baseline: 1.0068x; 1.0068x over previous
import jax
import jax.numpy as jnp
from jax.experimental import pallas as pl

_BM = 256


def _mm_kernel(xt_ref, w_ref, b_ref, o_ref):
    acc = jax.lax.dot_general(
        w_ref[...].astype(jnp.bfloat16), xt_ref[...],
        dimension_numbers=(((1,), (0,)), ((), ())),
        preferred_element_type=jnp.float32,
    )
    o_ref[...] = acc + b_ref[...]


def kernel(x, W, bias):
    n, k = W.shape
    b = x.shape[0]
    xt = x.T.astype(jnp.bfloat16)
    b2 = bias.reshape(n, 1)
    grid = (n // _BM,)
    yt = pl.pallas_call(
        _mm_kernel,
        grid=grid,
        in_specs=[
            pl.BlockSpec((k, b), lambda i: (0, 0)),
            pl.BlockSpec((_BM, k), lambda i: (i, 0)),
            pl.BlockSpec((_BM, 1), lambda i: (i, 0)),
        ],
        out_specs=pl.BlockSpec((_BM, b), lambda i: (i, 0)),
        out_shape=jax.ShapeDtypeStruct((n, b), jnp.float32),
    )(xt, W, b2)
    return yt.T


# speedup vs baseline: 1.0647x; 1.0575x over previous
import jax
import jax.numpy as jnp
from jax.experimental import pallas as pl

_BM = 256


def _mm_kernel(xb_ref, w_ref, b_ref, o_ref):
    acc = jax.lax.dot_general(
        xb_ref[...], w_ref[...].astype(jnp.bfloat16),
        dimension_numbers=(((1,), (1,)), ((), ())),
        preferred_element_type=jnp.float32,
    )
    o_ref[...] = acc + b_ref[...]


def kernel(x, W, bias):
    n, k = W.shape
    b = x.shape[0]
    xb = x.astype(jnp.bfloat16)
    b2 = bias.reshape(1, n)
    grid = (n // _BM,)
    out = pl.pallas_call(
        _mm_kernel,
        grid=grid,
        in_specs=[
            pl.BlockSpec((b, k), lambda i: (0, 0)),
            pl.BlockSpec((_BM, k), lambda i: (i, 0)),
            pl.BlockSpec((1, _BM), lambda i: (0, i)),
        ],
        out_specs=pl.BlockSpec((b, _BM), lambda i: (0, i)),
        out_shape=jax.ShapeDtypeStruct((b, n), jnp.float32),
    )(xb, W, b2)
    return out
